# baseline (speedup 1.0000x reference)
import functools

import jax
import jax.numpy as jnp
from jax import lax
from jax.experimental import pallas as pl
from jax.experimental.pallas import tpu as pltpu
from jax.experimental.pallas import tpu_sc as plsc

_NC = 2
_NS = 16
_NW = _NC * _NS
_L = 16

_MAX_NORM = 10.0
_EPS = 1e-7


def _rsqrt_nr(x, iters=2):
    i = lax.bitcast_convert_type(x, jnp.int32)
    i = jnp.int32(0x5F3759DF) - (i >> 1)
    y = lax.bitcast_convert_type(i, jnp.float32)
    for _ in range(iters):
        y = y * (1.5 - 0.5 * x * y * y)
    return y


def _clip_scale(ss):
    ss = jnp.maximum(ss, jnp.float32(_EPS * _EPS))
    return jnp.minimum(jnp.float32(1.0), jnp.float32(_MAX_NORM) * _rsqrt_nr(ss))


def _tree_sum(vals):
    vals = list(vals)
    while len(vals) > 1:
        vals = [a + b for a, b in zip(vals[::2], vals[1::2])]
    return vals[0]


def _splat(v, dtype=jnp.int32):
    return jnp.full((_L,), v, dtype)


@functools.cache
def _build(B, H, D, n_in, n_out):
    assert D == 2 * _L and B % (_NW * _L) == 0
    bpw = B // _NW
    ngrp = bpw // _L
    rpg = _L * H
    ipc = 2 * H
    nch = rpg // ipc
    och = bpw // 128
    assert nch * ipc == rpg and och * 128 == bpw and ipc <= 128

    mesh = plsc.VectorSubcoreMesh(
        core_axis_name="c", subcore_axis_name="s",
        num_cores=_NC, num_subcores=_NS)

    def body(inp_ref, oidx_ref, win_ref, wout_ref, o1_ref, o2_ref,
             idx_v, rows_v, oidx_v, orows_v, out_v, acc_v, sem_g, sem_o):
        wid = lax.axis_index("s") * _NC + lax.axis_index("c")
        base = wid * bpw

        pltpu.sync_copy(inp_ref.at[pl.ds(wid * ngrp * nch, ngrp * nch)], idx_v)
        pltpu.sync_copy(oidx_ref.at[wid], oidx_v)

        for c in range(och):
            pltpu.async_copy(wout_ref.at[oidx_v.at[c]],
                             orows_v.at[pl.ds(c * 128, 128)], sem_o)

        def fire(g, p):
            for c in range(nch):
                pltpu.async_copy(
                    win_ref.at[idx_v.at[g * nch + c]],
                    rows_v.at[p, pl.ds(c * ipc, ipc)], sem_g.at[p])

        def drain(g, p):
            for c in range(nch):
                pltpu.make_async_copy(
                    win_ref.at[idx_v.at[g * nch + c]],
                    rows_v.at[p, pl.ds(c * ipc, ipc)], sem_g.at[p]).wait()

        fire(0, 0)
        lanes = lax.iota(jnp.int32, _L)
        brow = lanes * H

        def gstep(g, _):
            p = lax.rem(g, 2)
            drain(g, p)

            @pl.when(g < ngrp - 1)
            def _prefetch():
                fire(g + 1, 1 - p)

            for d in range(D):
                acc_v[d, :] = jnp.zeros((_L,), jnp.float32)

            def lstep(l, _):
                rs = brow + l
                cols = [plsc.load_gather(rows_v.at[p], [rs, _splat(d)])
                        for d in range(D)]
                scale = _clip_scale(_tree_sum(c * c for c in cols))
                for d in range(D):
                    plsc.addupdate(acc_v.at[d], scale * cols[d])
                return 0

            lax.fori_loop(0, H, lstep, 0)

            for bb in range(_L):
                for h in range(2):
                    v = plsc.load_gather(acc_v, [lanes + h * _L, _splat(bb)])
                    out_v[g * _L + bb, pl.ds(h * _L, _L)] = v
            return 0

        lax.fori_loop(0, ngrp, gstep, 0)

        for c in range(och):
            pltpu.make_async_copy(wout_ref.at[oidx_v.at[c]],
                                  orows_v.at[pl.ds(c * 128, 128)], sem_o).wait()

        def ostep(g, _):
            rs = lanes + g * _L
            cols = [plsc.load_gather(orows_v, [rs, _splat(d)])
                    for d in range(D)]
            scale = _clip_scale(_tree_sum(c * c for c in cols))
            for d in range(D):
                plsc.store_scatter(orows_v, [rs, _splat(d)], scale * cols[d])
            return 0

        lax.fori_loop(0, ngrp, ostep, 0)

        pltpu.sync_copy(out_v, o1_ref.at[pl.ds(base, bpw)])
        pltpu.sync_copy(orows_v, o2_ref.at[pl.ds(base, bpw)])

    return pl.kernel(
        body,
        out_type=(jax.ShapeDtypeStruct((B, D), jnp.float32),
                  jax.ShapeDtypeStruct((B, D), jnp.float32)),
        mesh=mesh,
        compiler_params=pltpu.CompilerParams(
            use_tc_tiling_on_sc=False, needs_layout_passes=False),
        scratch_types=[
            pltpu.VMEM((ngrp * nch, ipc), jnp.int32),
            pltpu.VMEM((2, rpg, D), jnp.float32),
            pltpu.VMEM((bpw // 128, 128), jnp.int32),
            pltpu.VMEM((bpw, D), jnp.float32),
            pltpu.VMEM((bpw, D), jnp.float32),
            pltpu.VMEM((D, _L), jnp.float32),
            pltpu.SemaphoreType.DMA((2,)),
            pltpu.SemaphoreType.DMA,
        ],
    )


def kernel(input, output, W_in, W_out):
    B, H = input.shape
    n_in, D = W_in.shape
    n_out = W_out.shape[0]
    bpw = B // _NW
    fn = _build(B, H, D, n_in, n_out)
    iidx = input.astype(jnp.int32).reshape(-1, 2 * H)
    oidx = output.astype(jnp.int32).reshape(_NW, bpw // 128, 128)
    return fn(iidx, oidx, W_in, W_out)
